# baseline (speedup 1.0000x reference)
import functools

import jax
import jax.numpy as jnp
from jax import lax
from jax.experimental import pallas as pl
from jax.experimental.pallas import tpu as pltpu
from jax.experimental.pallas import tpu_sc as plsc

_NC = 2
_NS = 16
_LANES = 16
_TRASH = 1024
_B = 128
_G = 128
_NBUF = 4


def _gelu(x):
    return 0.5 * x * (1.0 + lax.erf(x * (2.0 ** -0.5)))


@functools.lru_cache(maxsize=None)
def _make_segsum(n, e_pad, h):
    half = n // _NC
    acc_rows = half + _TRASH
    acc_pad = ((acc_rows + _G - 1) // _G) * _G
    cpt = e_pad // _NS
    nch = cpt // _B
    k = _B // _G
    vi_iters = _B // _LANES
    per_row = _G // _LANES

    mesh = plsc.VectorSubcoreMesh(core_axis_name="c", subcore_axis_name="s")

    def body(maps_hbm, src_hbm, dst_hbm, out_hbm,
             src_v, dst_v, idx2d, rows_v, acc, *sems):
        isem = sems[0:_NBUF]
        gsem = sems[_NBUF:2 * _NBUF]
        ssem = sems[2 * _NBUF:3 * _NBUF]
        cid = lax.axis_index("c")
        tid = lax.axis_index("s")
        base = cid * half

        def _zb(i, c):
            rows_v[0, i // (h // _LANES),
                   pl.ds((i % (h // _LANES)) * _LANES, _LANES)] = (
                jnp.zeros((_LANES,), jnp.float32))
            return c
        lax.fori_loop(0, (_G * h) // _LANES, _zb, 0)

        nzch = acc_pad // _G
        def _zc(kk, c):
            ch = tid + _NS * kk
            @pl.when(ch < nzch)
            def _():
                pltpu.sync_copy(rows_v.at[0, pl.ds(0, _G)],
                                acc.at[pl.ds(ch * _G, _G)])
            return c
        lax.fori_loop(0, (nzch + _NS - 1) // _NS, _zc, 0)
        plsc.subcore_barrier()

        ebase = tid * cpt

        def _fire_idx(ci, b):
            eoff = ebase + ci * _B
            pltpu.async_copy(src_hbm.at[pl.ds(eoff, _B)], src_v.at[b], isem[b])
            pltpu.async_copy(dst_hbm.at[pl.ds(eoff, _B)], dst_v.at[b], isem[b])

        for b in range(_NBUF):
            if b < nch:
                _fire_idx(b, b)

        def _chunk(ci, c):
            for b in range(_NBUF):
                @pl.when((ci & (_NBUF - 1)) == b)
                def _():
                    pltpu.make_async_copy(
                        src_hbm.at[pl.ds(0, _B)], src_v.at[b], isem[b]).wait()
                    pltpu.make_async_copy(
                        dst_hbm.at[pl.ds(0, _B)], dst_v.at[b], isem[b]).wait()

                    @pl.when(ci >= _NBUF)
                    def _():
                        for j in range(k):
                            pltpu.make_async_copy(
                                rows_v.at[b, pl.ds(j * _G, _G)],
                                acc.at[pl.ds(0, _G)], ssem[b]).wait()

                    def _vi(i, cc):
                        d = dst_v[b, pl.ds(i * _LANES, _LANES)]
                        loc = d - base
                        oob = (loc < 0) | (loc >= half)
                        tr = half + ((i * _LANES + lax.iota(jnp.int32, _LANES)
                                      + tid * 64) & (_TRASH - 1))
                        idx2d[b, i // per_row,
                              pl.ds((i % per_row) * _LANES, _LANES)] = (
                            jnp.where(oob, tr, loc))
                        return cc
                    lax.fori_loop(0, vi_iters, _vi, 0)

                    cps = []
                    for j in range(k):
                        cps.append(pltpu.async_copy(
                            maps_hbm.at[src_v.at[b, pl.ds(j * _G, _G)]],
                            rows_v.at[b, pl.ds(j * _G, _G)], gsem[b]))
                    for cp in cps:
                        cp.wait()

                    for j in range(k):
                        pltpu.async_copy(rows_v.at[b, pl.ds(j * _G, _G)],
                                         acc.at[idx2d.at[b, j]], ssem[b],
                                         add=True)

                    @pl.when(ci + _NBUF < nch)
                    def _():
                        _fire_idx(ci + _NBUF, b)
            return c
        lax.fori_loop(0, nch, _chunk, 0)

        for b in range(_NBUF):
            if nch > b:
                for j in range(k):
                    pltpu.make_async_copy(
                        rows_v.at[b, pl.ds(j * _G, _G)],
                        acc.at[pl.ds(0, _G)], ssem[b]).wait()
        plsc.subcore_barrier()

        nwch = half // _G
        tail = half - nwch * _G
        def _wc(kk, c):
            ch = tid + _NS * kk
            @pl.when(ch < nwch)
            def _():
                pltpu.sync_copy(acc.at[pl.ds(ch * _G, _G)],
                                out_hbm.at[pl.ds(base + ch * _G, _G)])
            return c
        lax.fori_loop(0, (nwch + _NS - 1) // _NS, _wc, 0)
        if tail:
            @pl.when(tid == _NS - 1)
            def _():
                pltpu.sync_copy(acc.at[pl.ds(nwch * _G, tail)],
                                out_hbm.at[pl.ds(base + nwch * _G, tail)])

    return pl.kernel(
        body,
        out_type=jax.ShapeDtypeStruct((n, h), jnp.float32),
        mesh=mesh,
        scratch_types=[
            pltpu.VMEM((_NBUF, _B), jnp.int32),
            pltpu.VMEM((_NBUF, _B), jnp.int32),
            pltpu.VMEM((_NBUF, k, _G), jnp.int32),
            pltpu.VMEM((_NBUF, _B, h), jnp.float32),
            pltpu.VMEM_SHARED((acc_pad, h), jnp.float32),
        ] + [pltpu.SemaphoreType.DMA] * (3 * _NBUF),
        compiler_params=pltpu.CompilerParams(use_tc_tiling_on_sc=False),
    )


_ROWS = 5000


def _emb1(x, w, b):
    n, d_in = x.shape
    h = w.shape[1]
    def bodyfn(x_ref, w_ref, b_ref, o_ref):
        o_ref[...] = _gelu(
            jnp.dot(x_ref[...], w_ref[...], preferred_element_type=jnp.float32)
            + b_ref[...])
    return pl.pallas_call(
        bodyfn,
        grid=(n // _ROWS,),
        in_specs=[
            pl.BlockSpec((_ROWS, d_in), lambda i: (i, 0)),
            pl.BlockSpec((d_in, h), lambda i: (0, 0)),
            pl.BlockSpec((1, h), lambda i: (0, 0)),
        ],
        out_specs=pl.BlockSpec((_ROWS, h), lambda i: (i, 0)),
        out_shape=jax.ShapeDtypeStruct((n, h), jnp.float32),
    )(x, w, b.reshape(1, h))


def _layer(m, a, ws, wn):
    n, h = m.shape
    def bodyfn(m_ref, a_ref, ws_ref, wn_ref, o_ref):
        mm = m_ref[...]
        t = _gelu(
            jnp.dot(mm, ws_ref[...], preferred_element_type=jnp.float32)
            + jnp.dot(a_ref[...], wn_ref[...], preferred_element_type=jnp.float32))
        o_ref[...] = t + mm
    return pl.pallas_call(
        bodyfn,
        grid=(n // _ROWS,),
        in_specs=[
            pl.BlockSpec((_ROWS, h), lambda i: (i, 0)),
            pl.BlockSpec((_ROWS, h), lambda i: (i, 0)),
            pl.BlockSpec((h, h), lambda i: (0, 0)),
            pl.BlockSpec((h, h), lambda i: (0, 0)),
        ],
        out_specs=pl.BlockSpec((_ROWS, h), lambda i: (i, 0)),
        out_shape=jax.ShapeDtypeStruct((n, h), jnp.float32),
    )(m, a, ws, wn)


def _final(m, a, ws, wn, w2, b2):
    n, h = m.shape
    d_out = w2.shape[1]
    def bodyfn(m_ref, a_ref, ws_ref, wn_ref, w2_ref, b2_ref, o_ref):
        mm = m_ref[...]
        t = _gelu(
            jnp.dot(mm, ws_ref[...], preferred_element_type=jnp.float32)
            + jnp.dot(a_ref[...], wn_ref[...], preferred_element_type=jnp.float32)
        ) + mm
        o_ref[...] = jnp.tanh(
            jnp.dot(t, w2_ref[...], preferred_element_type=jnp.float32)
            + b2_ref[...])
    return pl.pallas_call(
        bodyfn,
        grid=(n // _ROWS,),
        in_specs=[
            pl.BlockSpec((_ROWS, h), lambda i: (i, 0)),
            pl.BlockSpec((_ROWS, h), lambda i: (i, 0)),
            pl.BlockSpec((h, h), lambda i: (0, 0)),
            pl.BlockSpec((h, h), lambda i: (0, 0)),
            pl.BlockSpec((h, d_out), lambda i: (0, 0)),
            pl.BlockSpec((1, d_out), lambda i: (0, 0)),
        ],
        out_specs=pl.BlockSpec((_ROWS, d_out), lambda i: (i, 0)),
        out_shape=jax.ShapeDtypeStruct((n, d_out), jnp.float32),
    )(m, a, ws, wn, w2, b2.reshape(1, d_out))


def kernel(x, edge_index, W_emb1, b_emb1, Ws0, Wn0, Ws1, Wn1, W_emb2, b_emb2):
    n = x.shape[0]
    e = edge_index.shape[1]
    h = W_emb1.shape[1]

    e_pad = -(-e // (_NS * _B)) * (_NS * _B)
    src = edge_index[0]
    dst = edge_index[1]
    if e_pad != e:
        pad = e_pad - e
        src = jnp.concatenate([src, jnp.zeros((pad,), jnp.int32)])
        dst = jnp.concatenate([dst, jnp.full((pad,), -1, jnp.int32)])

    segsum = _make_segsum(n, e_pad, h)

    maps0 = _emb1(x, W_emb1, b_emb1)
    agg0 = segsum(maps0, src, dst)
    maps1 = _layer(maps0, agg0, Ws0, Wn0)
    agg1 = segsum(maps1, src, dst)
    return _final(maps1, agg1, Ws1, Wn1, W_emb2, b_emb2)


# speedup vs baseline: 1.4430x; 1.4430x over previous
import functools

import jax
import jax.numpy as jnp
from jax import lax
from jax.experimental import pallas as pl
from jax.experimental.pallas import tpu as pltpu
from jax.experimental.pallas import tpu_sc as plsc

_NC = 2
_NS = 16
_LANES = 16
_TRASH = 1024
_B = 384
_G = 128
_NBUF = 2


def _gelu(x):
    return 0.5 * x * (1.0 + lax.erf(x * (2.0 ** -0.5)))


@functools.lru_cache(maxsize=None)
def _make_segsum(n, e_pad, h):
    half = n // _NC
    acc_rows = half + _TRASH
    acc_pad = ((acc_rows + _G - 1) // _G) * _G
    cpt = e_pad // _NS
    nch = cpt // _B
    k = _B // _G
    vi_iters = _B // _LANES
    per_row = _G // _LANES

    mesh = plsc.VectorSubcoreMesh(core_axis_name="c", subcore_axis_name="s")

    def body(maps_hbm, src_hbm, dst_hbm, out_hbm,
             src_v, dst_v, idx2d, rows_v, acc, *sems):
        isem = sems[0:_NBUF]
        gsem = sems[_NBUF:2 * _NBUF]
        ssem = sems[2 * _NBUF:3 * _NBUF]
        cid = lax.axis_index("c")
        tid = lax.axis_index("s")
        base = cid * half

        def _zb(i, c):
            rows_v[0, i // (h // _LANES),
                   pl.ds((i % (h // _LANES)) * _LANES, _LANES)] = (
                jnp.zeros((_LANES,), jnp.float32))
            return c
        lax.fori_loop(0, (_G * h) // _LANES, _zb, 0)

        nzch = acc_pad // _G
        def _zc(kk, c):
            ch = tid + _NS * kk
            @pl.when(ch < nzch)
            def _():
                pltpu.sync_copy(rows_v.at[0, pl.ds(0, _G)],
                                acc.at[pl.ds(ch * _G, _G)])
            return c
        lax.fori_loop(0, (nzch + _NS - 1) // _NS, _zc, 0)
        plsc.subcore_barrier()

        ebase = tid * cpt

        def _fire_idx(ci, b):
            eoff = ebase + ci * _B
            pltpu.async_copy(src_hbm.at[pl.ds(eoff, _B)], src_v.at[b], isem[b])
            pltpu.async_copy(dst_hbm.at[pl.ds(eoff, _B)], dst_v.at[b], isem[b])

        for b in range(_NBUF):
            if b < nch:
                _fire_idx(b, b)

        def _chunk(ci, c):
            for b in range(_NBUF):
                @pl.when((ci & (_NBUF - 1)) == b)
                def _():
                    pltpu.make_async_copy(
                        src_hbm.at[pl.ds(0, _B)], src_v.at[b], isem[b]).wait()
                    pltpu.make_async_copy(
                        dst_hbm.at[pl.ds(0, _B)], dst_v.at[b], isem[b]).wait()

                    @pl.when(ci >= _NBUF)
                    def _():
                        for j in range(k):
                            pltpu.make_async_copy(
                                rows_v.at[b, pl.ds(j * _G, _G)],
                                acc.at[pl.ds(0, _G)], ssem[b]).wait()

                    def _vi(i, cc):
                        d = dst_v[b, pl.ds(i * _LANES, _LANES)]
                        loc = d - base
                        oob = (loc < 0) | (loc >= half)
                        tr = half + ((i * _LANES + lax.iota(jnp.int32, _LANES)
                                      + tid * 64) & (_TRASH - 1))
                        idx2d[b, i // per_row,
                              pl.ds((i % per_row) * _LANES, _LANES)] = (
                            jnp.where(oob, tr, loc))
                        return cc
                    lax.fori_loop(0, vi_iters, _vi, 0)

                    cps = []
                    for j in range(k):
                        cps.append(pltpu.async_copy(
                            maps_hbm.at[src_v.at[b, pl.ds(j * _G, _G)]],
                            rows_v.at[b, pl.ds(j * _G, _G)], gsem[b]))
                    for cp in cps:
                        cp.wait()

                    for j in range(k):
                        pltpu.async_copy(rows_v.at[b, pl.ds(j * _G, _G)],
                                         acc.at[idx2d.at[b, j]], ssem[b],
                                         add=True)

                    @pl.when(ci + _NBUF < nch)
                    def _():
                        _fire_idx(ci + _NBUF, b)
            return c
        lax.fori_loop(0, nch, _chunk, 0)

        for b in range(_NBUF):
            if nch > b:
                for j in range(k):
                    pltpu.make_async_copy(
                        rows_v.at[b, pl.ds(j * _G, _G)],
                        acc.at[pl.ds(0, _G)], ssem[b]).wait()
        plsc.subcore_barrier()

        nwch = half // _G
        tail = half - nwch * _G
        def _wc(kk, c):
            ch = tid + _NS * kk
            @pl.when(ch < nwch)
            def _():
                pltpu.sync_copy(acc.at[pl.ds(ch * _G, _G)],
                                out_hbm.at[pl.ds(base + ch * _G, _G)])
            return c
        lax.fori_loop(0, (nwch + _NS - 1) // _NS, _wc, 0)
        if tail:
            @pl.when(tid == _NS - 1)
            def _():
                pltpu.sync_copy(acc.at[pl.ds(nwch * _G, tail)],
                                out_hbm.at[pl.ds(base + nwch * _G, tail)])

    return pl.kernel(
        body,
        out_type=jax.ShapeDtypeStruct((n, h), jnp.float32),
        mesh=mesh,
        scratch_types=[
            pltpu.VMEM((_NBUF, _B), jnp.int32),
            pltpu.VMEM((_NBUF, _B), jnp.int32),
            pltpu.VMEM((_NBUF, k, _G), jnp.int32),
            pltpu.VMEM((_NBUF, _B, h), jnp.float32),
            pltpu.VMEM_SHARED((acc_pad, h), jnp.float32),
        ] + [pltpu.SemaphoreType.DMA] * (3 * _NBUF),
        compiler_params=pltpu.CompilerParams(use_tc_tiling_on_sc=False),
    )


_ROWS = 5000


def _emb1(x, w, b):
    n, d_in = x.shape
    h = w.shape[1]
    def bodyfn(x_ref, w_ref, b_ref, o_ref):
        o_ref[...] = _gelu(
            jnp.dot(x_ref[...], w_ref[...], preferred_element_type=jnp.float32)
            + b_ref[...])
    return pl.pallas_call(
        bodyfn,
        grid=(n // _ROWS,),
        in_specs=[
            pl.BlockSpec((_ROWS, d_in), lambda i: (i, 0)),
            pl.BlockSpec((d_in, h), lambda i: (0, 0)),
            pl.BlockSpec((1, h), lambda i: (0, 0)),
        ],
        out_specs=pl.BlockSpec((_ROWS, h), lambda i: (i, 0)),
        out_shape=jax.ShapeDtypeStruct((n, h), jnp.float32),
    )(x, w, b.reshape(1, h))


def _layer(m, a, ws, wn):
    n, h = m.shape
    def bodyfn(m_ref, a_ref, ws_ref, wn_ref, o_ref):
        mm = m_ref[...]
        t = _gelu(
            jnp.dot(mm, ws_ref[...], preferred_element_type=jnp.float32)
            + jnp.dot(a_ref[...], wn_ref[...], preferred_element_type=jnp.float32))
        o_ref[...] = t + mm
    return pl.pallas_call(
        bodyfn,
        grid=(n // _ROWS,),
        in_specs=[
            pl.BlockSpec((_ROWS, h), lambda i: (i, 0)),
            pl.BlockSpec((_ROWS, h), lambda i: (i, 0)),
            pl.BlockSpec((h, h), lambda i: (0, 0)),
            pl.BlockSpec((h, h), lambda i: (0, 0)),
        ],
        out_specs=pl.BlockSpec((_ROWS, h), lambda i: (i, 0)),
        out_shape=jax.ShapeDtypeStruct((n, h), jnp.float32),
    )(m, a, ws, wn)


def _final(m, a, ws, wn, w2, b2):
    n, h = m.shape
    d_out = w2.shape[1]
    def bodyfn(m_ref, a_ref, ws_ref, wn_ref, w2_ref, b2_ref, o_ref):
        mm = m_ref[...]
        t = _gelu(
            jnp.dot(mm, ws_ref[...], preferred_element_type=jnp.float32)
            + jnp.dot(a_ref[...], wn_ref[...], preferred_element_type=jnp.float32)
        ) + mm
        o_ref[...] = jnp.tanh(
            jnp.dot(t, w2_ref[...], preferred_element_type=jnp.float32)
            + b2_ref[...])
    return pl.pallas_call(
        bodyfn,
        grid=(n // _ROWS,),
        in_specs=[
            pl.BlockSpec((_ROWS, h), lambda i: (i, 0)),
            pl.BlockSpec((_ROWS, h), lambda i: (i, 0)),
            pl.BlockSpec((h, h), lambda i: (0, 0)),
            pl.BlockSpec((h, h), lambda i: (0, 0)),
            pl.BlockSpec((h, d_out), lambda i: (0, 0)),
            pl.BlockSpec((1, d_out), lambda i: (0, 0)),
        ],
        out_specs=pl.BlockSpec((_ROWS, d_out), lambda i: (i, 0)),
        out_shape=jax.ShapeDtypeStruct((n, d_out), jnp.float32),
    )(m, a, ws, wn, w2, b2.reshape(1, d_out))


def kernel(x, edge_index, W_emb1, b_emb1, Ws0, Wn0, Ws1, Wn1, W_emb2, b_emb2):
    n = x.shape[0]
    e = edge_index.shape[1]
    h = W_emb1.shape[1]

    e_pad = -(-e // (_NS * _B)) * (_NS * _B)
    src = edge_index[0]
    dst = edge_index[1]
    if e_pad != e:
        pad = e_pad - e
        src = jnp.concatenate([src, jnp.zeros((pad,), jnp.int32)])
        dst = jnp.concatenate([dst, jnp.full((pad,), -1, jnp.int32)])

    segsum = _make_segsum(n, e_pad, h)

    maps0 = _emb1(x, W_emb1, b_emb1)
    agg0 = segsum(maps0, src, dst)
    maps1 = _layer(maps0, agg0, Ws0, Wn0)
    agg1 = segsum(maps1, src, dst)
    return _final(maps1, agg1, Ws1, Wn1, W_emb2, b_emb2)


# speedup vs baseline: 1.4666x; 1.0163x over previous
import functools

import jax
import jax.numpy as jnp
from jax import lax
from jax.experimental import pallas as pl
from jax.experimental.pallas import tpu as pltpu
from jax.experimental.pallas import tpu_sc as plsc

_NC = 2
_NS = 16
_LANES = 16
_TRASH = 1024
_B = 384
_G = 128
_NBUF = 2


def _gelu(x):
    return 0.5 * x * (1.0 + lax.erf(x * (2.0 ** -0.5)))


@functools.lru_cache(maxsize=None)
def _make_segsum(n, e_pad, h):
    half = n // _NC
    acc_rows = half + _TRASH
    acc_pad = ((acc_rows + _G - 1) // _G) * _G
    cpt = e_pad // _NS
    nch = cpt // _B
    k = _B // _G
    vi_iters = _B // _LANES
    per_row = _G // _LANES

    mesh = plsc.VectorSubcoreMesh(core_axis_name="c", subcore_axis_name="s")

    def body(maps_hbm, src_hbm, dst_hbm, out_hbm,
             src_v, dst_v, idx2d, rows_v, acc, *sems):
        isem = sems[0:_NBUF]
        gsem = sems[_NBUF:2 * _NBUF]
        ssem = sems[2 * _NBUF:3 * _NBUF]
        cid = lax.axis_index("c")
        tid = lax.axis_index("s")
        base = cid * half

        def _zb(i, c):
            rows_v[0, i // (h // _LANES),
                   pl.ds((i % (h // _LANES)) * _LANES, _LANES)] = (
                jnp.zeros((_LANES,), jnp.float32))
            return c
        lax.fori_loop(0, (_G * h) // _LANES, _zb, 0)

        nzch = acc_pad // _G
        def _zc(kk, c):
            ch = tid + _NS * kk
            @pl.when(ch < nzch)
            def _():
                pltpu.sync_copy(rows_v.at[0, pl.ds(0, _G)],
                                acc.at[pl.ds(ch * _G, _G)])
            return c
        lax.fori_loop(0, (nzch + _NS - 1) // _NS, _zc, 0)
        plsc.subcore_barrier()

        ebase = tid * cpt

        def _fire_idx(ci, b):
            eoff = ebase + ci * _B
            pltpu.async_copy(src_hbm.at[pl.ds(eoff, _B)], src_v.at[b], isem[b])
            pltpu.async_copy(dst_hbm.at[pl.ds(eoff, _B)], dst_v.at[b], isem[b])

        for b in range(_NBUF):
            if b < nch:
                _fire_idx(b, b)

        def _chunk(ci, c):
            for b in range(_NBUF):
                @pl.when((ci & (_NBUF - 1)) == b)
                def _():
                    pltpu.make_async_copy(
                        src_hbm.at[pl.ds(0, _B)], src_v.at[b], isem[b]).wait()
                    pltpu.make_async_copy(
                        dst_hbm.at[pl.ds(0, _B)], dst_v.at[b], isem[b]).wait()

                    @pl.when(ci >= _NBUF)
                    def _():
                        for j in range(k):
                            pltpu.make_async_copy(
                                rows_v.at[b, pl.ds(j * _G, _G)],
                                acc.at[pl.ds(0, _G)], ssem[b]).wait()

                    def _vi(i, cc):
                        d = dst_v[b, pl.ds(i * _LANES, _LANES)]
                        loc = d - base
                        oob = (loc < 0) | (loc >= half)
                        tr = half + ((i * _LANES + lax.iota(jnp.int32, _LANES)
                                      + tid * 64) & (_TRASH - 1))
                        idx2d[b, i // per_row,
                              pl.ds((i % per_row) * _LANES, _LANES)] = (
                            jnp.where(oob, tr, loc))
                        return cc
                    lax.fori_loop(0, vi_iters, _vi, 0)

                    cps = []
                    for j in range(k):
                        cps.append(pltpu.async_copy(
                            maps_hbm.at[src_v.at[b, pl.ds(j * _G, _G)]],
                            rows_v.at[b, pl.ds(j * _G, _G)], gsem[b]))
                    for cp in cps:
                        cp.wait()

                    for j in range(k):
                        pltpu.async_copy(rows_v.at[b, pl.ds(j * _G, _G)],
                                         acc.at[idx2d.at[b, j]], ssem[b],
                                         add=True)

                    @pl.when(ci + _NBUF < nch)
                    def _():
                        _fire_idx(ci + _NBUF, b)
            return c
        lax.fori_loop(0, nch, _chunk, 0)

        for b in range(_NBUF):
            if nch > b:
                for j in range(k):
                    pltpu.make_async_copy(
                        rows_v.at[b, pl.ds(j * _G, _G)],
                        acc.at[pl.ds(0, _G)], ssem[b]).wait()
        plsc.subcore_barrier()

        nwch = half // _G
        tail = half - nwch * _G
        def _wc(kk, c):
            ch = tid + _NS * kk
            @pl.when(ch < nwch)
            def _():
                pltpu.sync_copy(acc.at[pl.ds(ch * _G, _G)],
                                out_hbm.at[pl.ds(base + ch * _G, _G)])
            return c
        lax.fori_loop(0, (nwch + _NS - 1) // _NS, _wc, 0)
        if tail:
            @pl.when(tid == _NS - 1)
            def _():
                pltpu.sync_copy(acc.at[pl.ds(nwch * _G, tail)],
                                out_hbm.at[pl.ds(base + nwch * _G, tail)])

    return pl.kernel(
        body,
        out_type=jax.ShapeDtypeStruct((n, h), jnp.float32),
        mesh=mesh,
        scratch_types=[
            pltpu.VMEM((_NBUF, _B), jnp.int32),
            pltpu.VMEM((_NBUF, _B), jnp.int32),
            pltpu.VMEM((_NBUF, k, _G), jnp.int32),
            pltpu.VMEM((_NBUF, _B, h), jnp.float32),
            pltpu.VMEM_SHARED((acc_pad, h), jnp.float32),
        ] + [pltpu.SemaphoreType.DMA] * (3 * _NBUF),
        compiler_params=pltpu.CompilerParams(use_tc_tiling_on_sc=False),
    )


_ROWS = 10000


def _emb1(x, w, b):
    n, d_in = x.shape
    h = w.shape[1]
    def bodyfn(x_ref, w_ref, b_ref, o_ref):
        o_ref[...] = _gelu(
            jnp.dot(x_ref[...], w_ref[...], preferred_element_type=jnp.float32)
            + b_ref[...])
    return pl.pallas_call(
        bodyfn,
        grid=(n // _ROWS,),
        in_specs=[
            pl.BlockSpec((_ROWS, d_in), lambda i: (i, 0)),
            pl.BlockSpec((d_in, h), lambda i: (0, 0)),
            pl.BlockSpec((1, h), lambda i: (0, 0)),
        ],
        out_specs=pl.BlockSpec((_ROWS, h), lambda i: (i, 0)),
        out_shape=jax.ShapeDtypeStruct((n, h), jnp.float32),
    )(x, w, b.reshape(1, h))


def _layer(m, a, ws, wn):
    n, h = m.shape
    def bodyfn(m_ref, a_ref, ws_ref, wn_ref, o_ref):
        mm = m_ref[...]
        t = _gelu(
            jnp.dot(mm, ws_ref[...], preferred_element_type=jnp.float32)
            + jnp.dot(a_ref[...], wn_ref[...], preferred_element_type=jnp.float32))
        o_ref[...] = t + mm
    return pl.pallas_call(
        bodyfn,
        grid=(n // _ROWS,),
        in_specs=[
            pl.BlockSpec((_ROWS, h), lambda i: (i, 0)),
            pl.BlockSpec((_ROWS, h), lambda i: (i, 0)),
            pl.BlockSpec((h, h), lambda i: (0, 0)),
            pl.BlockSpec((h, h), lambda i: (0, 0)),
        ],
        out_specs=pl.BlockSpec((_ROWS, h), lambda i: (i, 0)),
        out_shape=jax.ShapeDtypeStruct((n, h), jnp.float32),
    )(m, a, ws, wn)


def _final(m, a, ws, wn, w2, b2):
    n, h = m.shape
    d_out = w2.shape[1]
    def bodyfn(m_ref, a_ref, ws_ref, wn_ref, w2_ref, b2_ref, o_ref):
        mm = m_ref[...]
        t = _gelu(
            jnp.dot(mm, ws_ref[...], preferred_element_type=jnp.float32)
            + jnp.dot(a_ref[...], wn_ref[...], preferred_element_type=jnp.float32)
        ) + mm
        o_ref[...] = jnp.tanh(
            jnp.dot(t, w2_ref[...], preferred_element_type=jnp.float32)
            + b2_ref[...])
    return pl.pallas_call(
        bodyfn,
        grid=(n // _ROWS,),
        in_specs=[
            pl.BlockSpec((_ROWS, h), lambda i: (i, 0)),
            pl.BlockSpec((_ROWS, h), lambda i: (i, 0)),
            pl.BlockSpec((h, h), lambda i: (0, 0)),
            pl.BlockSpec((h, h), lambda i: (0, 0)),
            pl.BlockSpec((h, d_out), lambda i: (0, 0)),
            pl.BlockSpec((1, d_out), lambda i: (0, 0)),
        ],
        out_specs=pl.BlockSpec((_ROWS, d_out), lambda i: (i, 0)),
        out_shape=jax.ShapeDtypeStruct((n, d_out), jnp.float32),
    )(m, a, ws, wn, w2, b2.reshape(1, d_out))


def kernel(x, edge_index, W_emb1, b_emb1, Ws0, Wn0, Ws1, Wn1, W_emb2, b_emb2):
    n = x.shape[0]
    e = edge_index.shape[1]
    h = W_emb1.shape[1]

    e_pad = -(-e // (_NS * _B)) * (_NS * _B)
    src = edge_index[0]
    dst = edge_index[1]
    if e_pad != e:
        pad = e_pad - e
        src = jnp.concatenate([src, jnp.zeros((pad,), jnp.int32)])
        dst = jnp.concatenate([dst, jnp.full((pad,), -1, jnp.int32)])

    segsum = _make_segsum(n, e_pad, h)

    maps0 = _emb1(x, W_emb1, b_emb1)
    agg0 = segsum(maps0, src, dst)
    maps1 = _layer(maps0, agg0, Ws0, Wn0)
    agg1 = segsum(maps1, src, dst)
    return _final(maps1, agg1, Ws1, Wn1, W_emb2, b_emb2)
